# baseline (speedup 1.0000x reference)
import functools

import jax
import jax.numpy as jnp
from jax import lax
from jax.experimental import pallas as pl
from jax.experimental.pallas import tpu as pltpu
from jax.experimental.pallas import tpu_sc as plsc

VOCAB = 100000
DIM = 64
B = 16384
L = 50

NC = 2
NS = 16
NW = NC * NS
CH = 256
TCB = B // CH
KPW = TCB // NW
NCH = L * KPW
TP = CH + 16


def _make_kernel():
  mesh = plsc.VectorSubcoreMesh(core_axis_name="c", subcore_axis_name="s")

  @functools.partial(
      pl.kernel,
      mesh=mesh,
      compiler_params=pltpu.CompilerParams(
          use_tc_tiling_on_sc=False, needs_layout_passes=False),
      out_type=jax.ShapeDtypeStruct((L * 8, B // 128, 8, 128), jnp.float32),
      scratch_types=[
          pltpu.VMEM((L, KPW * CH), jnp.int32),
          pltpu.VMEM((CH, DIM), jnp.float32),
          pltpu.VMEM((CH, DIM), jnp.float32),
          pltpu.VMEM((DIM, TP), jnp.float32),
          pltpu.VMEM((DIM, TP), jnp.float32),
          pltpu.SemaphoreType.DMA,
          pltpu.SemaphoreType.DMA,
      ],
  )
  def emb(table_hbm, xt_hbm, out_hbm, idx_v, g0, g1, t0, t1, gsem, wsem):
    gbufs = (g0, g1)
    tbufs = (t0, t1)
    wid = lax.axis_index("s") * NC + lax.axis_index("c")
    bcol0 = wid * (KPW * CH)

    pltpu.sync_copy(xt_hbm.at[:, pl.ds(bcol0, KPW * CH)], idx_v)

    lanes = lax.iota(jnp.int32, 16)
    drow = [lanes + q * 16 for q in range(4)]

    def idx_slice(j):
      l = j // KPW
      k = lax.rem(j, KPW)
      return idx_v.at[l, pl.ds(k * CH, CH)]

    def transpose(gbuf, tbuf):
      def browloop(it, carry):
        for s in range(4):
          b = it * 4 + s
          bcol = jnp.full((16,), 0, jnp.int32) + b
          vals = [gbuf[b, pl.ds(q * 16, 16)] for q in range(4)]
          for q in range(4):
            plsc.store_scatter(tbuf, [drow[q], bcol], vals[q])
        return carry

      lax.fori_loop(0, CH // 4, browloop, 0)

    for u in range(2):
      pltpu.async_copy(table_hbm.at[idx_slice(u)], gbufs[u], gsem)

    def chunk(j, gbuf, tbuf):
      l = j // KPW
      k = lax.rem(j, KPW)
      tcg = (wid * KPW + k) * 2
      pltpu.make_async_copy(table_hbm.at[idx_slice(j)], gbuf, gsem).wait()

      @pl.when(j >= 2)
      def _():
        for _w in range(16):
          pltpu.make_async_copy(
              tbuf.at[pl.ds(0, 8), pl.ds(0, 128)], out_hbm.at[0, 0],
              wsem).wait()

      transpose(gbuf, tbuf)
      for tr in range(8):
        for tcl in range(2):
          pltpu.async_copy(
              tbuf.at[pl.ds(tr * 8, 8), pl.ds(tcl * 128, 128)],
              out_hbm.at[l * 8 + tr, tcg + tcl], wsem)

      @pl.when(j + 2 < NCH)
      def _():
        pltpu.async_copy(table_hbm.at[idx_slice(j + 2)], gbuf, gsem)

    def body(gr, carry):
      for u in range(2):
        chunk(gr * 2 + u, gbufs[u], tbufs[u])
      return carry

    lax.fori_loop(0, NCH // 2, body, 0)

    for u in range(2):
      for _w in range(16):
        pltpu.make_async_copy(
            tbufs[u].at[pl.ds(0, 8), pl.ds(0, 128)], out_hbm.at[0, 0],
            wsem).wait()

  return emb


_emb = _make_kernel()


@jax.jit
def kernel(x, table):
  xt = x.T.astype(jnp.int32)
  q = _emb(table, xt)
  q5 = q.reshape(L, 8, B // 128, 8, 128)
  return q5.transpose(2, 4, 0, 1, 3).reshape(B, L, DIM)


# speedup vs baseline: 1.6140x; 1.6140x over previous
import functools

import jax
import jax.numpy as jnp
from jax import lax
from jax.experimental import pallas as pl
from jax.experimental.pallas import tpu as pltpu
from jax.experimental.pallas import tpu_sc as plsc

VOCAB = 100000
DIM = 64
B = 16384
L = 50

NC = 2
NS = 16
NW = NC * NS
CH = 256
TCB = B // CH
KPW = TCB // NW
NCH = L * KPW
TP = CH + 8


def _make_kernel():
  mesh = plsc.VectorSubcoreMesh(core_axis_name="c", subcore_axis_name="s")

  @functools.partial(
      pl.kernel,
      mesh=mesh,
      compiler_params=pltpu.CompilerParams(
          use_tc_tiling_on_sc=False, needs_layout_passes=False),
      out_type=jax.ShapeDtypeStruct((L * 8, B // 128, 8, 128), jnp.float32),
      scratch_types=[
          pltpu.VMEM((L, KPW * CH), jnp.int32),
          pltpu.VMEM((CH, DIM), jnp.float32),
          pltpu.VMEM((CH, DIM), jnp.float32),
          pltpu.VMEM((DIM, TP), jnp.float32),
          pltpu.VMEM((DIM, TP), jnp.float32),
          pltpu.SemaphoreType.DMA,
          pltpu.SemaphoreType.DMA,
      ],
  )
  def emb(table_hbm, xt_hbm, out_hbm, idx_v, g0, g1, t0, t1, gsem, wsem):
    gbufs = (g0, g1)
    tbufs = (t0, t1)
    wid = lax.axis_index("s") * NC + lax.axis_index("c")
    bcol0 = wid * (KPW * CH)

    pltpu.sync_copy(xt_hbm.at[:, pl.ds(bcol0, KPW * CH)], idx_v)

    lanes = lax.iota(jnp.int32, 16)
    drow = [lanes + q * 16 for q in range(4)]

    def idx_slice(j):
      l = j // KPW
      k = lax.rem(j, KPW)
      return idx_v.at[l, pl.ds(k * CH, CH)]

    def transpose(gbuf, tbuf):
      def browloop(it, carry):
        for s in range(4):
          b = it * 4 + s
          bcol = jnp.full((16,), 0, jnp.int32) + b
          vals = [gbuf[b, pl.ds(q * 16, 16)] for q in range(4)]
          for q in range(4):
            plsc.store_scatter(tbuf, [drow[q], bcol], vals[q])
        return carry

      lax.fori_loop(0, CH // 4, browloop, 0)

    for u in range(2):
      pltpu.async_copy(table_hbm.at[idx_slice(u)], gbufs[u], gsem)

    def chunk(j, gbuf, tbuf):
      l = j // KPW
      k = lax.rem(j, KPW)
      tcg = (wid * KPW + k) * 2
      pltpu.make_async_copy(table_hbm.at[idx_slice(j)], gbuf, gsem).wait()

      @pl.when(j >= 2)
      def _():
        for _w in range(16):
          pltpu.make_async_copy(
              tbuf.at[pl.ds(0, 8), pl.ds(0, 128)], out_hbm.at[0, 0],
              wsem).wait()

      transpose(gbuf, tbuf)
      for tr in range(8):
        for tcl in range(2):
          pltpu.async_copy(
              tbuf.at[pl.ds(tr * 8, 8), pl.ds(tcl * 128, 128)],
              out_hbm.at[l * 8 + tr, tcg + tcl], wsem)

      @pl.when(j + 2 < NCH)
      def _():
        pltpu.async_copy(table_hbm.at[idx_slice(j + 2)], gbuf, gsem)

    def body(gr, carry):
      for u in range(2):
        chunk(gr * 2 + u, gbufs[u], tbufs[u])
      return carry

    lax.fori_loop(0, NCH // 2, body, 0)

    for u in range(2):
      for _w in range(16):
        pltpu.make_async_copy(
            tbufs[u].at[pl.ds(0, 8), pl.ds(0, 128)], out_hbm.at[0, 0],
            wsem).wait()

  return emb


_emb = _make_kernel()


@jax.jit
def kernel(x, table):
  xt = x.T.astype(jnp.int32)
  q = _emb(table, xt)
  q5 = q.reshape(L, 8, B // 128, 8, 128)
  return q5.transpose(2, 4, 0, 1, 3).reshape(B, L, DIM)


# speedup vs baseline: 1.6868x; 1.0451x over previous
import functools

import jax
import jax.numpy as jnp
from jax import lax
from jax.experimental import pallas as pl
from jax.experimental.pallas import tpu as pltpu
from jax.experimental.pallas import tpu_sc as plsc

VOCAB = 100000
DIM = 64
B = 16384
L = 50

NC = 2
NS = 16
NW = NC * NS
CH = 128
TCB = B // CH
KPW = TCB // NW
NCH = L * KPW
TP = CH + 8


def _make_kernel():
  mesh = plsc.VectorSubcoreMesh(core_axis_name="c", subcore_axis_name="s")

  @functools.partial(
      pl.kernel,
      mesh=mesh,
      compiler_params=pltpu.CompilerParams(
          use_tc_tiling_on_sc=False, needs_layout_passes=False),
      out_type=jax.ShapeDtypeStruct((L * 8, TCB, 8, CH), jnp.float32),
      scratch_types=[
          pltpu.VMEM((L, KPW * CH), jnp.int32),
          pltpu.VMEM((CH, DIM), jnp.float32),
          pltpu.VMEM((CH, DIM), jnp.float32),
          pltpu.VMEM((CH, DIM), jnp.float32),
          pltpu.VMEM((CH, DIM), jnp.float32),
          pltpu.VMEM((DIM, TP), jnp.float32),
          pltpu.VMEM((DIM, TP), jnp.float32),
          pltpu.SemaphoreType.DMA,
          pltpu.SemaphoreType.DMA,
      ],
  )
  def emb(table_hbm, xt_hbm, out_hbm, idx_v, g0, g1, g2, g3, t0, t1,
          gsem, wsem):
    gbufs = (g0, g1, g2, g3)
    tbufs = (t0, t1)
    wid = lax.axis_index("s") * NC + lax.axis_index("c")
    bcol0 = wid * (KPW * CH)

    pltpu.sync_copy(xt_hbm.at[:, pl.ds(bcol0, KPW * CH)], idx_v)

    lanes = lax.iota(jnp.int32, 16)
    drow = [lanes + q * 16 for q in range(4)]

    def idx_slice(j):
      l = j // KPW
      k = lax.rem(j, KPW)
      return idx_v.at[l, pl.ds(k * CH, CH)]

    def transpose(gbuf, tbuf):
      def browloop(it, carry):
        for s in range(8):
          b = it * 8 + s
          bcol = jnp.full((16,), 0, jnp.int32) + b
          vals = [gbuf[b, pl.ds(q * 16, 16)] for q in range(4)]
          for q in range(4):
            plsc.store_scatter(tbuf, [drow[q], bcol], vals[q])
        return carry

      lax.fori_loop(0, CH // 8, browloop, 0)

    for u in range(3):
      pltpu.async_copy(table_hbm.at[idx_slice(u)], gbufs[u], gsem)

    def chunk(j, gbuf, gbuf_next, tbuf):
      l = j // KPW
      k = lax.rem(j, KPW)
      tcg = wid * KPW + k
      pltpu.make_async_copy(table_hbm.at[idx_slice(j)], gbuf, gsem).wait()

      @pl.when(j + 3 < NCH)
      def _():
        pltpu.async_copy(table_hbm.at[idx_slice(j + 3)], gbuf_next, gsem)

      @pl.when(j >= 2)
      def _():
        for _w in range(8):
          pltpu.make_async_copy(
              tbuf.at[pl.ds(0, 8), pl.ds(0, CH)], out_hbm.at[0, 0],
              wsem).wait()

      transpose(gbuf, tbuf)
      for tr in range(8):
        pltpu.async_copy(
            tbuf.at[pl.ds(tr * 8, 8), pl.ds(0, CH)],
            out_hbm.at[l * 8 + tr, tcg], wsem)

    def body(gr, carry):
      for u in range(4):
        j = gr * 4 + u
        chunk(j, gbufs[u], gbufs[(u + 3) % 4], tbufs[u % 2])
      return carry

    lax.fori_loop(0, NCH // 4, body, 0)

    for u in range(2):
      for _w in range(8):
        pltpu.make_async_copy(
            tbufs[u].at[pl.ds(0, 8), pl.ds(0, CH)], out_hbm.at[0, 0],
            wsem).wait()

  return emb


_emb = _make_kernel()


@jax.jit
def kernel(x, table):
  xt = x.T.astype(jnp.int32)
  q = _emb(table, xt)
  q5 = q.reshape(L, 8, TCB, 8, CH)
  return q5.transpose(2, 4, 0, 1, 3).reshape(B, L, DIM)
